# baseline (speedup 1.0000x reference)
import jax
import jax.numpy as jnp
from jax import lax
from jax.experimental import pallas as pl
from jax.experimental.pallas import tpu as pltpu
from jax.experimental.pallas import tpu_sc as plsc

VOCAB = 1_000_000
EMBED_DIM = 64
BATCH = 4096
HIST = 200

_L = 49152
_TC_GRID = -(-VOCAB // _L)


def _proj_body(x_ref, w_ref, o_ref):
    x = x_ref[...]
    w = w_ref[...]
    o_ref[...] = jnp.sum(x * w, axis=0)


def _project_table(e_t, w_col):
    return pl.pallas_call(
        _proj_body,
        grid=(_TC_GRID,),
        in_specs=[
            pl.BlockSpec((EMBED_DIM, _L), lambda i: (0, i)),
            pl.BlockSpec((EMBED_DIM, 1), lambda i: (0, 0)),
        ],
        out_specs=pl.BlockSpec((_L,), lambda i: (i,)),
        out_shape=jax.ShapeDtypeStruct((VOCAB,), jnp.float32),
    )(e_t, w_col)


_NW = 32
_B_PER_W = BATCH // _NW
_CHUNK = 128
_WINDOW = 32
_UNROLL = 4
_NSL = _B_PER_W // 16


_STAGE_C = 7808
_STAGE_K = 8
_STAGE_N = _STAGE_K * _STAGE_C
_STAGE_TAIL = VOCAB - 16 * _STAGE_N


def _bag_body(proj_hbm, idst_hbm, bias_hbm, out_hbm,
              idx_v, val_v, out_v, bias_v, proj_s, stage_a, stage_b,
              gsem, fsem, psem, isem):
    sid = lax.axis_index("s")
    wid = sid * 2 + lax.axis_index("c")
    base = wid * _B_PER_W

    pltpu.make_async_copy(idst_hbm.at[:, pl.ds(base, _B_PER_W)],
                          idx_v, isem).start()
    pltpu.make_async_copy(bias_hbm, bias_v, isem).start()

    bufs = (stage_a, stage_b)

    def _fetch(c):
        off = sid * _STAGE_N + c * _STAGE_C
        return pltpu.make_async_copy(proj_hbm.at[pl.ds(off, _STAGE_C)],
                                     bufs[c % 2], fsem)

    def _push(c):
        off = sid * _STAGE_N + c * _STAGE_C
        return pltpu.make_async_copy(bufs[c % 2],
                                     proj_s.at[pl.ds(off, _STAGE_C)], psem)

    _fetch(0).start()
    for c in range(_STAGE_K):
        _fetch(c).wait()
        _push(c).start()
        if c + 1 < _STAGE_K:
            if c >= 1:
                _push(c - 1).wait()
            _fetch(c + 1).start()
    _push(_STAGE_K - 1).wait()

    @pl.when(sid == 0)
    def _():
        toff = 16 * _STAGE_N
        pltpu.sync_copy(proj_hbm.at[pl.ds(toff, _STAGE_TAIL)],
                        stage_a.at[pl.ds(0, _STAGE_TAIL)])
        pltpu.sync_copy(stage_a.at[pl.ds(0, _STAGE_TAIL)],
                        proj_s.at[pl.ds(toff, _STAGE_TAIL)])

    pltpu.make_async_copy(idst_hbm.at[:, pl.ds(base, _B_PER_W)],
                          idx_v, isem).wait()
    pltpu.make_async_copy(bias_hbm, bias_v, isem).wait()
    plsc.subcore_barrier()

    def _gather(l):
        return pltpu.make_async_copy(proj_s.at[idx_v.at[l]],
                                     val_v.at[l], gsem)

    for j in range(_WINDOW):
        _gather(j).start()

    zero = jnp.zeros((16,), jnp.float32)

    def _acc_body(i, accs):
        for k in range(_UNROLL):
            l = i * _UNROLL + k
            _gather(l).wait()

            @pl.when(l < HIST - _WINDOW)
            def _(l=l):
                _gather(l + _WINDOW).start()

            accs = tuple(a + val_v[l, pl.ds(s * 16, 16)]
                         for s, a in enumerate(accs))
        return accs

    accs = lax.fori_loop(0, HIST // _UNROLL, _acc_body, (zero,) * _NSL)

    bias = bias_v[...]
    for s in range(_NSL):
        x = accs[s] + bias
        out_v[pl.ds(s * 16, 16)] = 1.0 / (1.0 + jnp.exp(-x))

    pltpu.sync_copy(out_v, out_hbm.at[pl.ds(base, _B_PER_W)])


def _bag(proj, ids_t, bias16):
    mesh = plsc.VectorSubcoreMesh(core_axis_name="c", subcore_axis_name="s")
    kern = pl.kernel(
        out_type=jax.ShapeDtypeStruct((BATCH,), jnp.float32),
        mesh=mesh,
        scratch_types=[
            pltpu.VMEM((HIST, _B_PER_W), jnp.int32),
            pltpu.VMEM((HIST, _B_PER_W), jnp.float32),
            pltpu.VMEM((_B_PER_W,), jnp.float32),
            pltpu.VMEM((16,), jnp.float32),
            pltpu.VMEM_SHARED((VOCAB,), jnp.float32),
            pltpu.VMEM((_STAGE_C,), jnp.float32),
            pltpu.VMEM((_STAGE_C,), jnp.float32),
            pltpu.SemaphoreType.DMA,
            pltpu.SemaphoreType.DMA,
            pltpu.SemaphoreType.DMA,
            pltpu.SemaphoreType.DMA,
        ],
    )(_bag_body)
    return kern(proj, ids_t, bias16)


def kernel(input_ids, embedding_matrix, linear_w, linear_b):
    e_t = embedding_matrix.T
    w_col = linear_w.reshape(EMBED_DIM, 1) / HIST
    proj = _project_table(e_t, w_col)
    ids_t = input_ids.astype(jnp.int32).T
    bias16 = jnp.broadcast_to(linear_b.astype(jnp.float32), (16,))
    out = _bag(proj, ids_t, bias16)
    return out.reshape(BATCH, 1)


# speedup vs baseline: 1.0166x; 1.0166x over previous
import jax
import jax.numpy as jnp
from jax import lax
from jax.experimental import pallas as pl
from jax.experimental.pallas import tpu as pltpu
from jax.experimental.pallas import tpu_sc as plsc

VOCAB = 1_000_000
EMBED_DIM = 64
BATCH = 4096
HIST = 200

_L = 32768
_TC_GRID = -(-VOCAB // _L)


def _proj_body(x_ref, w_ref, o_ref):
    x = x_ref[...]
    w = w_ref[...]
    o_ref[...] = jnp.sum(x * w, axis=0)


def _project_table(e_t, w_col):
    return pl.pallas_call(
        _proj_body,
        grid=(_TC_GRID,),
        in_specs=[
            pl.BlockSpec((EMBED_DIM, _L), lambda i: (0, i)),
            pl.BlockSpec((EMBED_DIM, 1), lambda i: (0, 0)),
        ],
        out_specs=pl.BlockSpec((_L,), lambda i: (i,)),
        out_shape=jax.ShapeDtypeStruct((VOCAB,), jnp.float32),
    )(e_t, w_col)


_NW = 32
_B_PER_W = BATCH // _NW
_CHUNK = 128
_WINDOW = 32
_UNROLL = 4
_NSL = _B_PER_W // 16


_STAGE_C = 7808
_STAGE_K = 8
_STAGE_N = _STAGE_K * _STAGE_C
_STAGE_TAIL = VOCAB - 16 * _STAGE_N


def _bag_body(proj_hbm, idst_hbm, bias_hbm, out_hbm,
              idx_v, val_v, out_v, bias_v, proj_s, stage_a, stage_b,
              gsem, fsem, psem, isem):
    sid = lax.axis_index("s")
    wid = sid * 2 + lax.axis_index("c")
    base = wid * _B_PER_W

    pltpu.make_async_copy(idst_hbm.at[:, pl.ds(base, _B_PER_W)],
                          idx_v, isem).start()
    pltpu.make_async_copy(bias_hbm, bias_v, isem).start()

    bufs = (stage_a, stage_b)

    def _fetch(c):
        off = sid * _STAGE_N + c * _STAGE_C
        return pltpu.make_async_copy(proj_hbm.at[pl.ds(off, _STAGE_C)],
                                     bufs[c % 2], fsem)

    def _push(c):
        off = sid * _STAGE_N + c * _STAGE_C
        return pltpu.make_async_copy(bufs[c % 2],
                                     proj_s.at[pl.ds(off, _STAGE_C)], psem)

    _fetch(0).start()
    for c in range(_STAGE_K):
        _fetch(c).wait()
        _push(c).start()
        if c + 1 < _STAGE_K:
            if c >= 1:
                _push(c - 1).wait()
            _fetch(c + 1).start()
    _push(_STAGE_K - 1).wait()

    @pl.when(sid == 0)
    def _():
        toff = 16 * _STAGE_N
        pltpu.sync_copy(proj_hbm.at[pl.ds(toff, _STAGE_TAIL)],
                        stage_a.at[pl.ds(0, _STAGE_TAIL)])
        pltpu.sync_copy(stage_a.at[pl.ds(0, _STAGE_TAIL)],
                        proj_s.at[pl.ds(toff, _STAGE_TAIL)])

    pltpu.make_async_copy(idst_hbm.at[:, pl.ds(base, _B_PER_W)],
                          idx_v, isem).wait()
    pltpu.make_async_copy(bias_hbm, bias_v, isem).wait()
    plsc.subcore_barrier()

    def _gather(l):
        return pltpu.make_async_copy(proj_s.at[idx_v.at[l]],
                                     val_v.at[l], gsem)

    for j in range(_WINDOW):
        _gather(j).start()

    zero = jnp.zeros((16,), jnp.float32)

    def _acc_body(i, accs):
        for k in range(_UNROLL):
            l = i * _UNROLL + k
            _gather(l).wait()

            @pl.when(l < HIST - _WINDOW)
            def _(l=l):
                _gather(l + _WINDOW).start()

            accs = tuple(a + val_v[l, pl.ds(s * 16, 16)]
                         for s, a in enumerate(accs))
        return accs

    accs = lax.fori_loop(0, HIST // _UNROLL, _acc_body, (zero,) * _NSL)

    bias = bias_v[...]
    for s in range(_NSL):
        x = accs[s] + bias
        out_v[pl.ds(s * 16, 16)] = 1.0 / (1.0 + jnp.exp(-x))

    pltpu.sync_copy(out_v, out_hbm.at[pl.ds(base, _B_PER_W)])


def _bag(proj, ids_t, bias16):
    mesh = plsc.VectorSubcoreMesh(core_axis_name="c", subcore_axis_name="s")
    kern = pl.kernel(
        out_type=jax.ShapeDtypeStruct((BATCH,), jnp.float32),
        mesh=mesh,
        scratch_types=[
            pltpu.VMEM((HIST, _B_PER_W), jnp.int32),
            pltpu.VMEM((HIST, _B_PER_W), jnp.float32),
            pltpu.VMEM((_B_PER_W,), jnp.float32),
            pltpu.VMEM((16,), jnp.float32),
            pltpu.VMEM_SHARED((VOCAB,), jnp.float32),
            pltpu.VMEM((_STAGE_C,), jnp.float32),
            pltpu.VMEM((_STAGE_C,), jnp.float32),
            pltpu.SemaphoreType.DMA,
            pltpu.SemaphoreType.DMA,
            pltpu.SemaphoreType.DMA,
            pltpu.SemaphoreType.DMA,
        ],
    )(_bag_body)
    return kern(proj, ids_t, bias16)


def kernel(input_ids, embedding_matrix, linear_w, linear_b):
    e_t = embedding_matrix.T
    w_col = linear_w.reshape(EMBED_DIM, 1) / HIST
    proj = _project_table(e_t, w_col)
    ids_t = input_ids.astype(jnp.int32).T
    bias16 = jnp.broadcast_to(linear_b.astype(jnp.float32), (16,))
    out = _bag(proj, ids_t, bias16)
    return out.reshape(BATCH, 1)


# speedup vs baseline: 1.0172x; 1.0005x over previous
import jax
import jax.numpy as jnp
from jax import lax
from jax.experimental import pallas as pl
from jax.experimental.pallas import tpu as pltpu
from jax.experimental.pallas import tpu_sc as plsc

VOCAB = 1_000_000
EMBED_DIM = 64
BATCH = 4096
HIST = 200

_L = 32768
_TC_GRID = -(-VOCAB // _L)


def _proj_body(x_ref, w_ref, o_ref):
    x = x_ref[...]
    w = w_ref[...]
    o_ref[...] = jnp.sum(x * w, axis=0)


def _project_table(e_t, w_col):
    return pl.pallas_call(
        _proj_body,
        grid=(_TC_GRID,),
        in_specs=[
            pl.BlockSpec((EMBED_DIM, _L), lambda i: (0, i)),
            pl.BlockSpec((EMBED_DIM, 1), lambda i: (0, 0)),
        ],
        out_specs=pl.BlockSpec((_L,), lambda i: (i,)),
        out_shape=jax.ShapeDtypeStruct((VOCAB,), jnp.float32),
    )(e_t, w_col)


_NW = 32
_B_PER_W = BATCH // _NW
_CHUNK = 128
_WINDOW = 32
_UNROLL = 4
_NSL = _B_PER_W // 16


_STAGE_C = 7808
_STAGE_K = 8
_STAGE_N = _STAGE_K * _STAGE_C
_STAGE_TAIL = VOCAB - 16 * _STAGE_N


def _bag_body(proj_hbm, idst_hbm, bias_hbm, out_hbm,
              idx_v, val_v, out_v, bias_v, proj_s, stage_a, stage_b,
              gsem, fsem, psem, isem):
    sid = lax.axis_index("s")
    wid = sid * 2 + lax.axis_index("c")
    base = wid * _B_PER_W

    pltpu.make_async_copy(idst_hbm.at[:, pl.ds(base, _B_PER_W)],
                          idx_v, isem).start()
    pltpu.make_async_copy(bias_hbm, bias_v, isem).start()

    bufs = (stage_a, stage_b)

    def _fetch(c):
        off = sid * _STAGE_N + c * _STAGE_C
        return pltpu.make_async_copy(proj_hbm.at[pl.ds(off, _STAGE_C)],
                                     bufs[c % 2], fsem)

    def _push(c):
        off = sid * _STAGE_N + c * _STAGE_C
        return pltpu.make_async_copy(bufs[c % 2],
                                     proj_s.at[pl.ds(off, _STAGE_C)], psem)

    _fetch(0).start()
    for c in range(_STAGE_K):
        _fetch(c).wait()
        if c >= 1:
            _push(c - 1).wait()
        _push(c).start()
        if c + 1 < _STAGE_K:
            _fetch(c + 1).start()
    _push(_STAGE_K - 1).wait()

    @pl.when(sid == 0)
    def _():
        toff = 16 * _STAGE_N
        pltpu.sync_copy(proj_hbm.at[pl.ds(toff, _STAGE_TAIL)],
                        stage_a.at[pl.ds(0, _STAGE_TAIL)])
        pltpu.sync_copy(stage_a.at[pl.ds(0, _STAGE_TAIL)],
                        proj_s.at[pl.ds(toff, _STAGE_TAIL)])

    pltpu.make_async_copy(idst_hbm.at[:, pl.ds(base, _B_PER_W)],
                          idx_v, isem).wait()
    pltpu.make_async_copy(bias_hbm, bias_v, isem).wait()
    plsc.subcore_barrier()

    def _gather(l):
        return pltpu.make_async_copy(proj_s.at[idx_v.at[l]],
                                     val_v.at[l], gsem)

    for j in range(_WINDOW):
        _gather(j).start()

    zero = jnp.zeros((16,), jnp.float32)

    def _acc_body(i, accs):
        for k in range(_UNROLL):
            l = i * _UNROLL + k
            _gather(l).wait()

            @pl.when(l < HIST - _WINDOW)
            def _(l=l):
                _gather(l + _WINDOW).start()

            accs = tuple(a + val_v[l, pl.ds(s * 16, 16)]
                         for s, a in enumerate(accs))
        return accs

    accs = lax.fori_loop(0, HIST // _UNROLL, _acc_body, (zero,) * _NSL)

    bias = bias_v[...]
    for s in range(_NSL):
        x = accs[s] + bias
        out_v[pl.ds(s * 16, 16)] = 1.0 / (1.0 + jnp.exp(-x))

    pltpu.sync_copy(out_v, out_hbm.at[pl.ds(base, _B_PER_W)])


def _bag(proj, ids_t, bias16):
    mesh = plsc.VectorSubcoreMesh(core_axis_name="c", subcore_axis_name="s")
    kern = pl.kernel(
        out_type=jax.ShapeDtypeStruct((BATCH,), jnp.float32),
        mesh=mesh,
        scratch_types=[
            pltpu.VMEM((HIST, _B_PER_W), jnp.int32),
            pltpu.VMEM((HIST, _B_PER_W), jnp.float32),
            pltpu.VMEM((_B_PER_W,), jnp.float32),
            pltpu.VMEM((16,), jnp.float32),
            pltpu.VMEM_SHARED((VOCAB,), jnp.float32),
            pltpu.VMEM((_STAGE_C,), jnp.float32),
            pltpu.VMEM((_STAGE_C,), jnp.float32),
            pltpu.SemaphoreType.DMA,
            pltpu.SemaphoreType.DMA,
            pltpu.SemaphoreType.DMA,
            pltpu.SemaphoreType.DMA,
        ],
    )(_bag_body)
    return kern(proj, ids_t, bias16)


def kernel(input_ids, embedding_matrix, linear_w, linear_b):
    e_t = embedding_matrix.T
    w_col = linear_w.reshape(EMBED_DIM, 1) / HIST
    proj = _project_table(e_t, w_col)
    ids_t = input_ids.astype(jnp.int32).T
    bias16 = jnp.broadcast_to(linear_b.astype(jnp.float32), (16,))
    out = _bag(proj, ids_t, bias16)
    return out.reshape(BATCH, 1)
